# Initial kernel scaffold; baseline (speedup 1.0000x reference)
import functools

import jax
import jax.numpy as jnp
from jax import lax
from jax.experimental import pallas as pl
from jax.experimental.pallas import tpu as pltpu
from jax.experimental.pallas import tpu_sc as plsc

_NC = 2
_NS = 16
_NW = _NC * _NS
_L = 16

_B = 2000
_CHUNK = 80


def _sc_sqdist(z_hbm, w_hbm, ai_hbm, aj_hbm, d2_hbm, ii, jj, zr, wr, d2v, sem):
    E = d2_hbm.shape[0]
    epw = E // _NW
    nblk = epw // _B
    nch = _B // _CHUNK

    wid = lax.axis_index("s") * _NC + lax.axis_index("c")
    base = wid * epw

    lane = lax.iota(jnp.int32, (_L,))

    def block_body(b, carry):
        off = base + b * _B
        pltpu.sync_copy(ai_hbm.at[pl.ds(off, _B)], ii)
        pltpu.sync_copy(aj_hbm.at[pl.ds(off, _B)], jj)
        descs = []
        for k in range(nch):
            s = k * _CHUNK
            descs.append(pltpu.async_copy(
                z_hbm.at[ii.at[pl.ds(s, _CHUNK)]], zr.at[pl.ds(s, _CHUNK)], sem))
            descs.append(pltpu.async_copy(
                w_hbm.at[jj.at[pl.ds(s, _CHUNK)]], wr.at[pl.ds(s, _CHUNK)], sem))
        for dsc in descs:
            dsc.wait()

        def group_body(g, carry2):
            e16 = g * _L + lane
            acc = jnp.zeros((_L,), jnp.float32)
            for d in range(16):
                dvec = jnp.full((_L,), d, jnp.int32)
                zc = plsc.load_gather(zr, [e16, dvec])
                wc = plsc.load_gather(wr, [e16, dvec])
                t = zc - wc
                acc = acc + t * t
            d2v[pl.ds(g * _L, _L)] = acc
            return carry2

        lax.fori_loop(0, _B // _L, group_body, 0, unroll=False)
        pltpu.sync_copy(d2v, d2_hbm.at[pl.ds(off, _B)])
        return carry

    lax.fori_loop(0, nblk, block_body, 0, unroll=False)


def _tc_reduce_body(bias_ref, d2_ref, out_ref):
    i = pl.program_id(0)

    @pl.when(i == 0)
    def _():
        out_ref[0, 0] = 0.0

    d2 = d2_ref[...]
    pdist = jnp.sqrt(d2) + 1e-8
    x = 10.0 * (bias_ref[0] - pdist)
    term = (jnp.maximum(x, 0.0) + jnp.log1p(jnp.exp(-jnp.abs(x)))) / 10.0
    out_ref[0, 0] += jnp.sum(term)


def kernel(latent_z, latent_w, bias, analytical_i, analytical_j):
    E = analytical_i.shape[0]

    sc_call = pl.kernel(
        _sc_sqdist,
        out_type=jax.ShapeDtypeStruct((E,), jnp.float32),
        mesh=plsc.VectorSubcoreMesh(core_axis_name="c", subcore_axis_name="s"),
        scratch_types=[
            pltpu.VMEM((_B,), jnp.int32),
            pltpu.VMEM((_B,), jnp.int32),
            pltpu.VMEM((_B, 16), jnp.float32),
            pltpu.VMEM((_B, 16), jnp.float32),
            pltpu.VMEM((_B,), jnp.float32),
            pltpu.SemaphoreType.DMA,
        ],
    )
    d2 = sc_call(latent_z, latent_w, analytical_i, analytical_j)

    rows, cols = 3125, 1024
    blk_rows = 625
    d2m = d2.reshape(rows, cols)
    out = pl.pallas_call(
        _tc_reduce_body,
        out_shape=jax.ShapeDtypeStruct((1, 1), jnp.float32),
        grid=(rows // blk_rows,),
        in_specs=[
            pl.BlockSpec(memory_space=pltpu.SMEM),
            pl.BlockSpec((blk_rows, cols), lambda i: (i, 0)),
        ],
        out_specs=pl.BlockSpec((1, 1), lambda i: (0, 0)),
    )(bias, d2m)
    return out[0, 0]


# speedup vs baseline: 23.7119x; 23.7119x over previous
import functools

import jax
import jax.numpy as jnp
from jax import lax
from jax.experimental import pallas as pl
from jax.experimental.pallas import tpu as pltpu
from jax.experimental.pallas import tpu_sc as plsc

_NC = 2
_NS = 16
_NW = _NC * _NS
_L = 16

_B = 2000
_CHUNK = 80


def _sc_sqdist(z_hbm, w_hbm, ai_hbm, aj_hbm, d2_hbm, ii, jj, zr, wr, d2v, sem):
    E = d2_hbm.shape[0]
    epw = E // _NW
    nblk = epw // _B
    nch = _B // _CHUNK

    wid = lax.axis_index("s") * _NC + lax.axis_index("c")
    base = wid * epw

    lane = lax.iota(jnp.int32, _L)

    def block_body(b, carry):
        off = base + b * _B
        pltpu.sync_copy(ai_hbm.at[pl.ds(off, _B)], ii)
        pltpu.sync_copy(aj_hbm.at[pl.ds(off, _B)], jj)
        descs = []
        for k in range(nch):
            s = k * _CHUNK
            descs.append(pltpu.async_copy(
                z_hbm.at[ii.at[pl.ds(s, _CHUNK)]], zr.at[pl.ds(s, _CHUNK)], sem))
            descs.append(pltpu.async_copy(
                w_hbm.at[jj.at[pl.ds(s, _CHUNK)]], wr.at[pl.ds(s, _CHUNK)], sem))
        for dsc in descs:
            dsc.wait()

        def group_body(g, carry2):
            e16 = g * _L + lane
            acc = jnp.zeros((_L,), jnp.float32)
            for d in range(16):
                dvec = jnp.full((_L,), d, jnp.int32)
                zc = plsc.load_gather(zr, [e16, dvec])
                wc = plsc.load_gather(wr, [e16, dvec])
                t = zc - wc
                acc = acc + t * t
            d2v[pl.ds(g * _L, _L)] = acc
            return carry2

        lax.fori_loop(0, _B // _L, group_body, 0, unroll=False)
        pltpu.sync_copy(d2v, d2_hbm.at[pl.ds(off, _B)])
        return carry

    lax.fori_loop(0, nblk, block_body, 0, unroll=False)


def _tc_reduce_body(bias_ref, d2_ref, out_ref):
    i = pl.program_id(0)

    @pl.when(i == 0)
    def _():
        out_ref[0, 0] = 0.0

    d2 = d2_ref[...]
    pdist = jnp.sqrt(d2) + 1e-8
    x = 10.0 * (bias_ref[0] - pdist)
    term = (jnp.maximum(x, 0.0) + jnp.log1p(jnp.exp(-jnp.abs(x)))) / 10.0
    out_ref[0, 0] += jnp.sum(term)


def kernel(latent_z, latent_w, bias, analytical_i, analytical_j):
    E = analytical_i.shape[0]

    sc_call = pl.kernel(
        _sc_sqdist,
        out_type=jax.ShapeDtypeStruct((E,), jnp.float32),
        mesh=plsc.VectorSubcoreMesh(core_axis_name="c", subcore_axis_name="s"),
        scratch_types=[
            pltpu.VMEM((_B,), jnp.int32),
            pltpu.VMEM((_B,), jnp.int32),
            pltpu.VMEM((_B, 16), jnp.float32),
            pltpu.VMEM((_B, 16), jnp.float32),
            pltpu.VMEM((_B,), jnp.float32),
            pltpu.SemaphoreType.DMA,
        ],
        compiler_params=pltpu.CompilerParams(
            needs_layout_passes=False, use_tc_tiling_on_sc=False),
    )
    d2 = sc_call(latent_z, latent_w, analytical_i, analytical_j)

    rows, cols = 1600, E // 1600
    blk_rows = 200
    d2m = d2.reshape(rows, cols)
    out = pl.pallas_call(
        _tc_reduce_body,
        out_shape=jax.ShapeDtypeStruct((1, 1), jnp.float32),
        grid=(rows // blk_rows,),
        in_specs=[
            pl.BlockSpec(memory_space=pltpu.SMEM),
            pl.BlockSpec((blk_rows, cols), lambda i: (i, 0)),
        ],
        out_specs=pl.BlockSpec(memory_space=pltpu.SMEM),
    )(bias, d2m)
    return out[0, 0]


# speedup vs baseline: 26.6552x; 1.1241x over previous
import functools

import jax
import jax.numpy as jnp
from jax import lax
from jax.experimental import pallas as pl
from jax.experimental.pallas import tpu as pltpu
from jax.experimental.pallas import tpu_sc as plsc

_NC = 2
_NS = 16
_NW = _NC * _NS
_L = 16

_B = 1000
_CHUNK = 40


def _sc_sqdist(z_hbm, w_hbm, ai_hbm, aj_hbm, d2_hbm,
               ii0, jj0, zr0, wr0, ii1, jj1, zr1, wr1, d2v,
               sem0, sem1):
    E = d2_hbm.shape[0]
    epw = E // _NW
    nblk = epw // _B
    nch = _B // _CHUNK
    bufs = ((ii0, jj0, zr0, wr0, sem0), (ii1, jj1, zr1, wr1, sem1))

    wid = lax.axis_index("s") * _NC + lax.axis_index("c")
    base = wid * epw

    lane = lax.iota(jnp.int32, _L)

    def stage_and_fire(b, buf):
        ii, jj, zr, wr, sem = buf
        off = base + b * _B
        pltpu.sync_copy(ai_hbm.at[pl.ds(off, _B)], ii)
        pltpu.sync_copy(aj_hbm.at[pl.ds(off, _B)], jj)
        for k in range(nch):
            s = k * _CHUNK
            pltpu.async_copy(
                z_hbm.at[ii.at[pl.ds(s, _CHUNK)]], zr.at[pl.ds(s, _CHUNK)], sem)
            pltpu.async_copy(
                w_hbm.at[jj.at[pl.ds(s, _CHUNK)]], wr.at[pl.ds(s, _CHUNK)], sem)

    def drain(buf):
        ii, jj, zr, wr, sem = buf
        for k in range(nch):
            s = k * _CHUNK
            pltpu.make_async_copy(
                z_hbm.at[ii.at[pl.ds(s, _CHUNK)]], zr.at[pl.ds(s, _CHUNK)], sem
            ).wait()
            pltpu.make_async_copy(
                w_hbm.at[jj.at[pl.ds(s, _CHUNK)]], wr.at[pl.ds(s, _CHUNK)], sem
            ).wait()

    def compute(b, buf):
        ii, jj, zr, wr, sem = buf
        off = base + b * _B

        def group_body(g, carry2):
            e16 = g * _L + lane
            acc = jnp.zeros((_L,), jnp.float32)
            for d in range(16):
                dvec = jnp.full((_L,), d, jnp.int32)
                zc = plsc.load_gather(zr, [e16, dvec])
                wc = plsc.load_gather(wr, [e16, dvec])
                t = zc - wc
                acc = acc + t * t
            d2v[pl.ds(g * _L, _L)] = acc
            return carry2

        lax.fori_loop(0, _B // _L, group_body, 0, unroll=False)
        pltpu.sync_copy(d2v, d2_hbm.at[pl.ds(off, _B)])

    stage_and_fire(0, bufs[0])

    def super_body(sstep, carry):
        for half in range(2):
            b = 2 * sstep + half
            nxt = bufs[1 - half]

            @pl.when(b + 1 < nblk)
            def _():
                stage_and_fire(b + 1, nxt)

            drain(bufs[half])
            compute(b, bufs[half])
        return carry

    lax.fori_loop(0, nblk // 2, super_body, 0, unroll=False)


def _tc_reduce_body(bias_ref, d2_ref, out_ref):
    i = pl.program_id(0)

    @pl.when(i == 0)
    def _():
        out_ref[0, 0] = 0.0

    d2 = d2_ref[...]
    pdist = jnp.sqrt(d2) + 1e-8
    x = 10.0 * (bias_ref[0] - pdist)
    term = (jnp.maximum(x, 0.0) + jnp.log1p(jnp.exp(-jnp.abs(x)))) / 10.0
    out_ref[0, 0] += jnp.sum(term)


def kernel(latent_z, latent_w, bias, analytical_i, analytical_j):
    E = analytical_i.shape[0]

    sc_call = pl.kernel(
        _sc_sqdist,
        out_type=jax.ShapeDtypeStruct((E,), jnp.float32),
        mesh=plsc.VectorSubcoreMesh(core_axis_name="c", subcore_axis_name="s"),
        scratch_types=[
            pltpu.VMEM((_B,), jnp.int32),
            pltpu.VMEM((_B,), jnp.int32),
            pltpu.VMEM((_B, 16), jnp.float32),
            pltpu.VMEM((_B, 16), jnp.float32),
            pltpu.VMEM((_B,), jnp.int32),
            pltpu.VMEM((_B,), jnp.int32),
            pltpu.VMEM((_B, 16), jnp.float32),
            pltpu.VMEM((_B, 16), jnp.float32),
            pltpu.VMEM((_B,), jnp.float32),
            pltpu.SemaphoreType.DMA,
            pltpu.SemaphoreType.DMA,
        ],
        compiler_params=pltpu.CompilerParams(
            needs_layout_passes=False, use_tc_tiling_on_sc=False),
    )
    d2 = sc_call(latent_z, latent_w, analytical_i, analytical_j)

    rows, cols = 1600, E // 1600
    blk_rows = 200
    d2m = d2.reshape(rows, cols)
    out = pl.pallas_call(
        _tc_reduce_body,
        out_shape=jax.ShapeDtypeStruct((1, 1), jnp.float32),
        grid=(rows // blk_rows,),
        in_specs=[
            pl.BlockSpec(memory_space=pltpu.SMEM),
            pl.BlockSpec((blk_rows, cols), lambda i: (i, 0)),
        ],
        out_specs=pl.BlockSpec(memory_space=pltpu.SMEM),
    )(bias, d2m)
    return out[0, 0]


# speedup vs baseline: 53.9918x; 2.0256x over previous
import functools

import jax
import jax.numpy as jnp
from jax import lax
from jax.experimental import pallas as pl
from jax.experimental.pallas import tpu as pltpu
from jax.experimental.pallas import tpu_sc as plsc

_NC = 2
_NS = 16
_NW = _NC * _NS
_L = 16

_B = 1000
_CHUNK = 40


def _sc_sqdist(z_hbm, w_hbm, ai_hbm, aj_hbm, d2_hbm,
               ii0, jj0, zr0, wr0, ii1, jj1, zr1, wr1, d2v,
               sem0, sem1):
    E = d2_hbm.shape[0]
    epw = E // _NW
    nblk = epw // _B
    nch = _B // _CHUNK
    bufs = ((ii0, jj0, zr0, wr0, sem0), (ii1, jj1, zr1, wr1, sem1))

    wid = lax.axis_index("s") * _NC + lax.axis_index("c")
    base = wid * epw

    lane = lax.iota(jnp.int32, _L)

    def stage_and_fire(b, buf):
        ii, jj, zr, wr, sem = buf
        off = base + b * _B
        pltpu.sync_copy(ai_hbm.at[pl.ds(off, _B)], ii)
        pltpu.sync_copy(aj_hbm.at[pl.ds(off, _B)], jj)
        for k in range(nch):
            s = k * _CHUNK
            pltpu.async_copy(
                z_hbm.at[ii.at[pl.ds(s, _CHUNK)]], zr.at[pl.ds(s, _CHUNK)], sem)
            pltpu.async_copy(
                w_hbm.at[jj.at[pl.ds(s, _CHUNK)]], wr.at[pl.ds(s, _CHUNK)], sem)

    def drain(buf):
        ii, jj, zr, wr, sem = buf
        for k in range(nch):
            s = k * _CHUNK
            pltpu.make_async_copy(
                z_hbm.at[ii.at[pl.ds(s, _CHUNK)]], zr.at[pl.ds(s, _CHUNK)], sem
            ).wait()
            pltpu.make_async_copy(
                w_hbm.at[jj.at[pl.ds(s, _CHUNK)]], wr.at[pl.ds(s, _CHUNK)], sem
            ).wait()

    def compute(b, buf):
        ii, jj, zr, wr, sem = buf
        off = base + b * _B

        def group_body(g, carry2):
            e16 = g * _L + lane
            acc = jnp.zeros((_L,), jnp.float32)
            for d in range(16):
                dvec = jnp.bitwise_and(lane + d, _L - 1)
                zc = plsc.load_gather(zr, [e16, dvec])
                wc = plsc.load_gather(wr, [e16, dvec])
                t = zc - wc
                acc = acc + t * t
            d2v[pl.ds(g * _L, _L)] = acc
            return carry2

        lax.fori_loop(0, _B // _L, group_body, 0, unroll=False)
        pltpu.sync_copy(d2v, d2_hbm.at[pl.ds(off, _B)])

    stage_and_fire(0, bufs[0])

    def super_body(sstep, carry):
        for half in range(2):
            b = 2 * sstep + half
            nxt = bufs[1 - half]

            @pl.when(b + 1 < nblk)
            def _():
                stage_and_fire(b + 1, nxt)

            drain(bufs[half])
            compute(b, bufs[half])
        return carry

    lax.fori_loop(0, nblk // 2, super_body, 0, unroll=False)


def _tc_reduce_body(bias_ref, d2_ref, out_ref):
    i = pl.program_id(0)

    @pl.when(i == 0)
    def _():
        out_ref[0, 0] = 0.0

    d2 = d2_ref[...]
    pdist = jnp.sqrt(d2) + 1e-8
    x = 10.0 * (bias_ref[0] - pdist)
    term = (jnp.maximum(x, 0.0) + jnp.log1p(jnp.exp(-jnp.abs(x)))) / 10.0
    out_ref[0, 0] += jnp.sum(term)


def kernel(latent_z, latent_w, bias, analytical_i, analytical_j):
    E = analytical_i.shape[0]

    sc_call = pl.kernel(
        _sc_sqdist,
        out_type=jax.ShapeDtypeStruct((E,), jnp.float32),
        mesh=plsc.VectorSubcoreMesh(core_axis_name="c", subcore_axis_name="s"),
        scratch_types=[
            pltpu.VMEM((_B,), jnp.int32),
            pltpu.VMEM((_B,), jnp.int32),
            pltpu.VMEM((_B, 16), jnp.float32),
            pltpu.VMEM((_B, 16), jnp.float32),
            pltpu.VMEM((_B,), jnp.int32),
            pltpu.VMEM((_B,), jnp.int32),
            pltpu.VMEM((_B, 16), jnp.float32),
            pltpu.VMEM((_B, 16), jnp.float32),
            pltpu.VMEM((_B,), jnp.float32),
            pltpu.SemaphoreType.DMA,
            pltpu.SemaphoreType.DMA,
        ],
        compiler_params=pltpu.CompilerParams(
            needs_layout_passes=False, use_tc_tiling_on_sc=False),
    )
    d2 = sc_call(latent_z, latent_w, analytical_i, analytical_j)

    rows, cols = 1600, E // 1600
    blk_rows = 200
    d2m = d2.reshape(rows, cols)
    out = pl.pallas_call(
        _tc_reduce_body,
        out_shape=jax.ShapeDtypeStruct((1, 1), jnp.float32),
        grid=(rows // blk_rows,),
        in_specs=[
            pl.BlockSpec(memory_space=pltpu.SMEM),
            pl.BlockSpec((blk_rows, cols), lambda i: (i, 0)),
        ],
        out_specs=pl.BlockSpec(memory_space=pltpu.SMEM),
    )(bias, d2m)
    return out[0, 0]


# speedup vs baseline: 61.7926x; 1.1445x over previous
import functools

import jax
import jax.numpy as jnp
from jax import lax
from jax.experimental import pallas as pl
from jax.experimental.pallas import tpu as pltpu
from jax.experimental.pallas import tpu_sc as plsc

_NC = 2
_NS = 16
_NW = _NC * _NS
_L = 16

_B = 1000
_CHUNK = 1000


def _sc_sqdist(z_hbm, w_hbm, ai_hbm, aj_hbm, d2_hbm,
               ii0, jj0, zr0, wr0, ii1, jj1, zr1, wr1, d2v,
               sem0, sem1):
    E = d2_hbm.shape[0]
    epw = E // _NW
    nblk = epw // _B
    nch = _B // _CHUNK
    bufs = ((ii0, jj0, zr0, wr0, sem0), (ii1, jj1, zr1, wr1, sem1))

    wid = lax.axis_index("s") * _NC + lax.axis_index("c")
    base = wid * epw

    lane = lax.iota(jnp.int32, _L)

    def stage_and_fire(b, buf):
        ii, jj, zr, wr, sem = buf
        off = base + b * _B
        pltpu.sync_copy(ai_hbm.at[pl.ds(off, _B)], ii)
        pltpu.sync_copy(aj_hbm.at[pl.ds(off, _B)], jj)
        for k in range(nch):
            s = k * _CHUNK
            pltpu.async_copy(
                z_hbm.at[ii.at[pl.ds(s, _CHUNK)]], zr.at[pl.ds(s, _CHUNK)], sem)
            pltpu.async_copy(
                w_hbm.at[jj.at[pl.ds(s, _CHUNK)]], wr.at[pl.ds(s, _CHUNK)], sem)

    def drain(buf):
        ii, jj, zr, wr, sem = buf
        for k in range(nch):
            s = k * _CHUNK
            pltpu.make_async_copy(
                z_hbm.at[ii.at[pl.ds(s, _CHUNK)]], zr.at[pl.ds(s, _CHUNK)], sem
            ).wait()
            pltpu.make_async_copy(
                w_hbm.at[jj.at[pl.ds(s, _CHUNK)]], wr.at[pl.ds(s, _CHUNK)], sem
            ).wait()

    def compute(b, buf):
        ii, jj, zr, wr, sem = buf
        off = base + b * _B

        def group_body(g, carry2):
            e16 = g * _L + lane
            acc = jnp.zeros((_L,), jnp.float32)
            for d in range(16):
                dvec = jnp.bitwise_and(lane + d, _L - 1)
                zc = plsc.load_gather(zr, [e16, dvec])
                wc = plsc.load_gather(wr, [e16, dvec])
                t = zc - wc
                acc = acc + t * t
            d2v[pl.ds(g * _L, _L)] = acc
            return carry2

        lax.fori_loop(0, _B // _L, group_body, 0, unroll=False)
        pltpu.sync_copy(d2v, d2_hbm.at[pl.ds(off, _B)])

    stage_and_fire(0, bufs[0])

    def super_body(sstep, carry):
        for half in range(2):
            b = 2 * sstep + half
            nxt = bufs[1 - half]

            @pl.when(b + 1 < nblk)
            def _():
                stage_and_fire(b + 1, nxt)

            drain(bufs[half])
            compute(b, bufs[half])
        return carry

    lax.fori_loop(0, nblk // 2, super_body, 0, unroll=False)


def _tc_reduce_body(bias_ref, d2_ref, out_ref):
    i = pl.program_id(0)

    @pl.when(i == 0)
    def _():
        out_ref[0, 0] = 0.0

    d2 = d2_ref[...]
    pdist = jnp.sqrt(d2) + 1e-8
    x = 10.0 * (bias_ref[0] - pdist)
    term = (jnp.maximum(x, 0.0) + jnp.log1p(jnp.exp(-jnp.abs(x)))) / 10.0
    out_ref[0, 0] += jnp.sum(term)


def kernel(latent_z, latent_w, bias, analytical_i, analytical_j):
    E = analytical_i.shape[0]

    sc_call = pl.kernel(
        _sc_sqdist,
        out_type=jax.ShapeDtypeStruct((E,), jnp.float32),
        mesh=plsc.VectorSubcoreMesh(core_axis_name="c", subcore_axis_name="s"),
        scratch_types=[
            pltpu.VMEM((_B,), jnp.int32),
            pltpu.VMEM((_B,), jnp.int32),
            pltpu.VMEM((_B, 16), jnp.float32),
            pltpu.VMEM((_B, 16), jnp.float32),
            pltpu.VMEM((_B,), jnp.int32),
            pltpu.VMEM((_B,), jnp.int32),
            pltpu.VMEM((_B, 16), jnp.float32),
            pltpu.VMEM((_B, 16), jnp.float32),
            pltpu.VMEM((_B,), jnp.float32),
            pltpu.SemaphoreType.DMA,
            pltpu.SemaphoreType.DMA,
        ],
        compiler_params=pltpu.CompilerParams(
            needs_layout_passes=False, use_tc_tiling_on_sc=False),
    )
    d2 = sc_call(latent_z, latent_w, analytical_i, analytical_j)

    rows, cols = 1600, E // 1600
    blk_rows = 200
    d2m = d2.reshape(rows, cols)
    out = pl.pallas_call(
        _tc_reduce_body,
        out_shape=jax.ShapeDtypeStruct((1, 1), jnp.float32),
        grid=(rows // blk_rows,),
        in_specs=[
            pl.BlockSpec(memory_space=pltpu.SMEM),
            pl.BlockSpec((blk_rows, cols), lambda i: (i, 0)),
        ],
        out_specs=pl.BlockSpec(memory_space=pltpu.SMEM),
    )(bias, d2m)
    return out[0, 0]
